# Initial kernel scaffold; baseline (speedup 1.0000x reference)
import jax
import jax.numpy as jnp
from jax.experimental import pallas as pl


def kernel(x):
    raise NotImplementedError("write your pallas kernel here")


# speedup vs baseline: 1.9008x; 1.9008x over previous
import jax
import jax.numpy as jnp
import numpy as np
from jax.experimental import pallas as pl

HI = jax.lax.Precision.HIGHEST
T = 4096
N1, N2 = 128, 32
R = 64


def _topk_gather_kernel(ac_ref, x4_ref, out_ref):
    AC = ac_ref[...]
    x4 = x4_ref[...]

    bits = jax.lax.bitcast_convert_type(AC, jnp.int32)
    keys = jnp.where(bits < 0, jnp.int32(0x7FFFFFFF) ^ bits, bits)
    lane = jax.lax.broadcasted_iota(jnp.int32, (R, T), 1)
    neg_inf = jnp.int32(-2147483648)

    i_id = jax.lax.broadcasted_iota(jnp.int32, (R, N1, N1), 1)
    i_src = jax.lax.broadcasted_iota(jnp.int32, (R, N1, N1), 2)
    j_src = jax.lax.broadcasted_iota(jnp.int32, (R, N2, N2), 1)
    j_id = jax.lax.broadcasted_iota(jnp.int32, (R, N2, N2), 2)

    acc = None
    for k in range(3):
        m = jnp.max(keys, axis=1, keepdims=True)
        idx = jnp.min(jnp.where(keys == m, lane, jnp.int32(T)), axis=1,
                      keepdims=True)
        if k < 2:
            keys = jnp.where(lane == idx, neg_inf, keys)
        a = (idx // N2)[:, :, None]
        b = (idx % N2)[:, :, None]
        Oa = (i_src == i_id - a).astype(jnp.float32)
        Oa1 = (i_src == i_id - a - 1).astype(jnp.float32)
        Qh = (j_src == j_id - b).astype(jnp.float32)
        Ql = (j_src == j_id - b + N2).astype(jnp.float32)
        g_hi = jax.lax.dot_general(Oa, x4, (((2,), (1,)), ((0,), (0,))),
                                   precision=HI)
        g_lo = jax.lax.dot_general(Oa1, x4, (((2,), (1,)), ((0,), (0,))),
                                   precision=HI)
        s_hi = jax.lax.dot_general(g_hi, Qh, (((2,), (1,)), ((0,), (0,))),
                                   precision=HI)
        s_lo = jax.lax.dot_general(g_lo, Ql, (((2,), (1,)), ((0,), (0,))),
                                   precision=HI)
        g = s_hi + s_lo
        acc = g if acc is None else acc + g
    out_ref[...] = acc * jnp.float32(1.0 / 3.0)


def kernel(x):
    B, C, _ = x.shape
    fft_x = jnp.fft.rfft(x, axis=-1)
    auto_corr = jnp.fft.irfft(fft_x * jnp.conj(fft_x), n=T, axis=-1)

    S = B * C
    ac2 = auto_corr.reshape(S, T)
    x4 = x.reshape(S, N1, N2)
    out = pl.pallas_call(
        _topk_gather_kernel,
        grid=(S // R,),
        in_specs=[
            pl.BlockSpec((R, T), lambda i: (i, 0)),
            pl.BlockSpec((R, N1, N2), lambda i: (i, 0, 0)),
        ],
        out_specs=pl.BlockSpec((R, N1, N2), lambda i: (i, 0, 0)),
        out_shape=jax.ShapeDtypeStruct((S, N1, N2), jnp.float32),
    )(ac2, x4)
    return out.reshape(B, C, T)
